# Initial kernel scaffold; baseline (speedup 1.0000x reference)
import jax
import jax.numpy as jnp
from jax.experimental import pallas as pl


def kernel(hidden_states, position_embeddings, Wq, Wk, Wv, Wo, gq, gk, kron_a, kron_b, sketch_scale, W1, b1, W2, b2):
    raise NotImplementedError("write your pallas kernel here")


# speedup vs baseline: 6.1940x; 6.1940x over previous
import functools
import math

import jax
import jax.numpy as jnp
from jax.experimental import pallas as pl
from jax.experimental.pallas import tpu as pltpu

B, S, HID = 1, 4096, 4096
NH, NKV, HD = 32, 8, 128
SK, TOPK = 640, 2048
RA, CA = 20, 128
RB, CB = 32, 256
GQ = NH // NKV

_NEG = -1e30


def _dot(a, b, trans_b=False):
    dn = (((1,), (1 if trans_b else 0,)), ((), ()))
    return jax.lax.dot_general(a, b, dn, precision=jax.lax.Precision.HIGHEST,
                               preferred_element_type=jnp.float32)


def _rank_body(lcol_ref, lrow_ref, tk_ref, rw_ref):
    i = pl.program_id(0)
    bm = lcol_ref.shape[0]
    vj = lcol_ref[...]
    vi = lrow_ref[...]
    irow = i * bm + jax.lax.broadcasted_iota(jnp.int32, (bm, 1), 0)
    icol = jax.lax.broadcasted_iota(jnp.int32, (1, S), 1)
    gt = (vi > vj).astype(jnp.float32)
    eqlt = ((vi == vj) & (icol < irow)).astype(jnp.float32)
    rank = jnp.sum(gt + eqlt, axis=1, keepdims=True)
    tk = (rank < float(TOPK)).astype(jnp.float32)
    tk_ref[...] = tk
    rw_ref[...] = jax.nn.sigmoid(vj) * (1.0 - tk)


def _rank(logits_col, logits_row):
    bm = 512
    return pl.pallas_call(
        _rank_body,
        grid=(S // bm,),
        in_specs=[
            pl.BlockSpec((bm, 1), lambda i: (i, 0)),
            pl.BlockSpec((1, S), lambda i: (0, 0)),
        ],
        out_specs=[
            pl.BlockSpec((bm, 1), lambda i: (i, 0)),
            pl.BlockSpec((bm, 1), lambda i: (i, 0)),
        ],
        out_shape=[
            jax.ShapeDtypeStruct((S, 1), jnp.float32),
            jax.ShapeDtypeStruct((S, 1), jnp.float32),
        ],
    )(logits_col, logits_row)


def _mm_body(x_ref, w_ref, o_ref):
    @pl.when(pl.program_id(2) == 0)
    def _():
        o_ref[...] = jnp.zeros_like(o_ref)

    o_ref[...] += _dot(x_ref[...], w_ref[...])


def _matmul(x, w, bm, bn, bk):
    m, k = x.shape
    _, n = w.shape
    return pl.pallas_call(
        _mm_body,
        grid=(m // bm, n // bn, k // bk),
        in_specs=[
            pl.BlockSpec((bm, bk), lambda i, j, kk: (i, kk)),
            pl.BlockSpec((bk, bn), lambda i, j, kk: (kk, j)),
        ],
        out_specs=pl.BlockSpec((bm, bn), lambda i, j, kk: (i, j)),
        out_shape=jax.ShapeDtypeStruct((m, n), jnp.float32),
        compiler_params=pltpu.CompilerParams(
            dimension_semantics=("parallel", "parallel", "arbitrary")),
    )(x, w)


def _mm2_body(x_ref, wk_ref, wv_ref, ok_ref, ov_ref):
    @pl.when(pl.program_id(1) == 0)
    def _():
        ok_ref[...] = jnp.zeros_like(ok_ref)
        ov_ref[...] = jnp.zeros_like(ov_ref)

    xb = x_ref[...]
    ok_ref[...] += _dot(xb, wk_ref[...])
    ov_ref[...] += _dot(xb, wv_ref[...])


def _matmul_kv(x, Wk, Wv):
    bm, bk = 1024, 512
    n = NKV * HD
    return pl.pallas_call(
        _mm2_body,
        grid=(S // bm, HID // bk),
        in_specs=[
            pl.BlockSpec((bm, bk), lambda i, kk: (i, kk)),
            pl.BlockSpec((bk, n), lambda i, kk: (kk, 0)),
            pl.BlockSpec((bk, n), lambda i, kk: (kk, 0)),
        ],
        out_specs=[
            pl.BlockSpec((bm, n), lambda i, kk: (i, 0)),
            pl.BlockSpec((bm, n), lambda i, kk: (i, 0)),
        ],
        out_shape=[
            jax.ShapeDtypeStruct((S, n), jnp.float32),
            jax.ShapeDtypeStruct((S, n), jnp.float32),
        ],
        compiler_params=pltpu.CompilerParams(
            dimension_semantics=("parallel", "arbitrary")),
    )(x, Wk, Wv)


def _rmsnorm(x3, g):
    ms = jnp.mean(x3 * x3, axis=-1, keepdims=True)
    return x3 * jax.lax.rsqrt(ms + 1e-6) * g


def _rope(x3, cosb, sinb):
    x1 = x3[..., : HD // 2]
    x2 = x3[..., HD // 2:]
    h1 = x1 * cosb[..., : HD // 2] - x2 * sinb[..., : HD // 2]
    h2 = x2 * cosb[..., HD // 2:] + x1 * sinb[..., HD // 2:]
    return jnp.concatenate([h1, h2], axis=-1)


def _epi_body2(q_ref, k_ref, v_ref, cos_ref, sin_ref, gq_ref, gk_ref, rw_ref,
               qo_ref, ko_ref, kv_ref):
    bm = q_ref.shape[0]
    cosb = cos_ref[...][:, None, :]
    sinb = sin_ref[...][:, None, :]
    qn = _rmsnorm(q_ref[...], gq_ref[...])
    qo_ref[...] = _rope(qn, cosb, sinb).reshape(bm, NH * HD)
    kn = _rmsnorm(k_ref[...], gk_ref[...])
    ko_ref[...] = _rope(kn, cosb, sinb).reshape(bm, NKV * HD)
    rw = rw_ref[...]
    kv_ref[:, : NKV * HD] = kn.reshape(bm, NKV * HD) * rw
    kv_ref[:, NKV * HD:] = v_ref[...].reshape(bm, NKV * HD) * rw


def _epilogue(q, k, v, cos, sin, gq, gk, restw):
    bm = 256
    return pl.pallas_call(
        _epi_body2,
        grid=(S // bm,),
        in_specs=[
            pl.BlockSpec((bm, NH, HD), lambda i: (i, 0, 0)),
            pl.BlockSpec((bm, NKV, HD), lambda i: (i, 0, 0)),
            pl.BlockSpec((bm, NKV, HD), lambda i: (i, 0, 0)),
            pl.BlockSpec((bm, HD), lambda i: (i, 0)),
            pl.BlockSpec((bm, HD), lambda i: (i, 0)),
            pl.BlockSpec((1, 1, HD), lambda i: (0, 0, 0)),
            pl.BlockSpec((1, 1, HD), lambda i: (0, 0, 0)),
            pl.BlockSpec((bm, 1), lambda i: (i, 0)),
        ],
        out_specs=[
            pl.BlockSpec((bm, NH * HD), lambda i: (i, 0)),
            pl.BlockSpec((bm, NKV * HD), lambda i: (i, 0)),
            pl.BlockSpec((bm, 2 * NKV * HD), lambda i: (i, 0)),
        ],
        out_shape=[
            jax.ShapeDtypeStruct((S, NH * HD), jnp.float32),
            jax.ShapeDtypeStruct((S, NKV * HD), jnp.float32),
            jax.ShapeDtypeStruct((S, 2 * NKV * HD), jnp.float32),
        ],
    )(q.reshape(S, NH, HD), k.reshape(S, NKV, HD), v.reshape(S, NKV, HD),
      cos, sin, gq.reshape(1, 1, HD), gk.reshape(1, 1, HD), restw)


def _sk_body(m_ref, x_ref, o_ref):
    @pl.when(pl.program_id(0) == 0)
    def _():
        o_ref[...] = jnp.zeros_like(o_ref)

    o_ref[...] += _dot(m_ref[...], x_ref[...])


def _sketch(M, kvw):
    bk = 1024
    return pl.pallas_call(
        _sk_body,
        grid=(S // bk,),
        in_specs=[
            pl.BlockSpec((SK, bk), lambda kk: (0, kk)),
            pl.BlockSpec((bk, 2 * NKV * HD), lambda kk: (kk, 0)),
        ],
        out_specs=pl.BlockSpec((SK, 2 * NKV * HD), lambda kk: (0, 0)),
        out_shape=jax.ShapeDtypeStruct((SK, 2 * NKV * HD), jnp.float32),
        compiler_params=pltpu.CompilerParams(
            dimension_semantics=("arbitrary",)),
    )(M, kvw)


def _attn_body(q_ref, k_ref, v_ref, sk_ref, sv_ref, tk_ref, o_ref, *, bm, bkk):
    i = pl.program_id(1)
    scale = 1.0 / math.sqrt(HD)
    qpos = i * bm + jax.lax.broadcasted_iota(jnp.int32, (bm, 1), 0)
    qpos_f = qpos.astype(jnp.float32)

    ratio = jnp.float32((CA * CB) / SK)
    stimes = jax.lax.broadcasted_iota(
        jnp.int32, (1, SK), 1).astype(jnp.float32) * ratio
    smask = stimes <= qpos_f
    skb = sk_ref[...]
    svb = sv_ref[...]

    for hh in range(GQ):
        q = q_ref[:, hh * HD:(hh + 1) * HD] * scale

        def body(j, carry):
            m, l, acc = carry
            kb = k_ref[pl.ds(j * bkk, bkk), :]
            vb = v_ref[pl.ds(j * bkk, bkk), :]
            tk = tk_ref[:, pl.ds(j * bkk, bkk)]
            kpos = j * bkk + jax.lax.broadcasted_iota(jnp.int32, (1, bkk), 1)
            okb = (tk > 0.0) & (kpos <= qpos)
            lg = _dot(q, kb, trans_b=True)
            lg = jnp.where(okb, lg, _NEG)
            mn = jnp.maximum(m, jnp.max(lg, axis=1, keepdims=True))
            p = jnp.where(okb, jnp.exp(lg - mn), 0.0)
            alpha = jnp.exp(m - mn)
            l = l * alpha + jnp.sum(p, axis=1, keepdims=True)
            acc = acc * alpha + _dot(p, vb)
            return mn, l, acc

        m0 = jnp.full((bm, 1), _NEG, jnp.float32)
        l0 = jnp.zeros((bm, 1), jnp.float32)
        a0 = jnp.zeros((bm, HD), jnp.float32)
        m, l, acc = jax.lax.fori_loop(0, i + 1, body, (m0, l0, a0))

        lg = _dot(q, skb, trans_b=True)
        lg = jnp.where(smask, lg, _NEG)
        mn = jnp.maximum(m, jnp.max(lg, axis=1, keepdims=True))
        p = jnp.where(smask, jnp.exp(lg - mn), 0.0)
        alpha = jnp.exp(m - mn)
        l = l * alpha + jnp.sum(p, axis=1, keepdims=True)
        acc = acc * alpha + _dot(p, svb)
        o_ref[:, hh * HD:(hh + 1) * HD] = acc / l


def _attention(q_rope, k_rope, v, sketch_kv, topk_row):
    bm = 512
    bkk = 512
    return pl.pallas_call(
        functools.partial(_attn_body, bm=bm, bkk=bkk),
        grid=(NKV, S // bm),
        in_specs=[
            pl.BlockSpec((bm, GQ * HD), lambda h, i: (i, h)),
            pl.BlockSpec((S, HD), lambda h, i: (0, h)),
            pl.BlockSpec((S, HD), lambda h, i: (0, h)),
            pl.BlockSpec((SK, HD), lambda h, i: (0, h)),
            pl.BlockSpec((SK, HD), lambda h, i: (0, NKV + h)),
            pl.BlockSpec((1, S), lambda h, i: (0, 0)),
        ],
        out_specs=pl.BlockSpec((bm, GQ * HD), lambda h, i: (i, h)),
        out_shape=jax.ShapeDtypeStruct((S, NH * HD), jnp.float32),
        compiler_params=pltpu.CompilerParams(
            dimension_semantics=("arbitrary", "arbitrary")),
    )(q_rope, k_rope, v, sketch_kv, sketch_kv, topk_row)


def _causal_mask(rows, cols):
    r = jnp.arange(rows)[:, None]
    c = jnp.arange(cols)[None, :]
    return (c < ((r + 1) * cols) // rows).astype(jnp.float32)


def _fold_sketch_matrix(kron_a, kron_b, sketch_scale):
    cka = kron_a * _causal_mask(RA, CA)
    ckb = kron_b * _causal_mask(RB, CB)
    nca = S // CB
    m4 = cka[None, :, :nca, None] * ckb[:, None, None, :]
    return m4.reshape(RB * RA, nca * CB) * sketch_scale[0]


def _freqs(position_embeddings):
    inv = 1.0 / (10000.0 ** (jnp.arange(0, HD, 2, dtype=jnp.float32) / HD))
    f = position_embeddings[0].astype(jnp.float32)[:, None] * inv[None, :]
    emb = jnp.concatenate([f, f], axis=-1)
    return jnp.cos(emb), jnp.sin(emb)


def kernel(hidden_states, position_embeddings, Wq, Wk, Wv, Wo, gq, gk,
           kron_a, kron_b, sketch_scale, W1, b1, W2, b2):
    x = hidden_states.reshape(S, HID)
    cos, sin = _freqs(position_embeddings)
    M = _fold_sketch_matrix(kron_a, kron_b, sketch_scale)

    imp_logits = jnp.tanh(hidden_states @ W1 + b1) @ W2 + b2
    imp_logits = imp_logits - math.log(S / SK)
    logits = imp_logits.reshape(S, 1)
    topk_col, restw = _rank(logits, logits.reshape(1, S))
    q = _matmul(x, Wq, bm=1024, bn=1024, bk=1024)
    k, v = _matmul_kv(x, Wk, Wv)
    q_rope, k_rope, kvw = _epilogue(q, k, v, cos, sin, gq, gk, restw)
    sketch_kv = _sketch(M, kvw)
    attn = _attention(q_rope, k_rope, v, sketch_kv, topk_col.reshape(1, S))
    out = _matmul(attn, Wo, bm=1024, bn=1024, bk=1024)
    return out.reshape(B, S, HID)


# speedup vs baseline: 23.4709x; 3.7893x over previous
import functools
import math

import jax
import jax.numpy as jnp
from jax import lax
from jax.experimental import pallas as pl
from jax.experimental.pallas import tpu as pltpu
from jax.experimental.pallas import tpu_sc as plsc

B, S, HID = 1, 4096, 4096
NH, NKV, HD = 32, 8, 128
SK, TOPK = 640, 2048
RA, CA = 20, 128
RB, CB = 32, 256
GQ = NH // NKV

_NEG = -1e30


def _dot(a, b, trans_b=False):
    dn = (((1,), (1 if trans_b else 0,)), ((), ()))
    return jax.lax.dot_general(a, b, dn, precision=jax.lax.Precision.DEFAULT,
                               preferred_element_type=jnp.float32)


def _rank_body(lcol_ref, lrow_ref, tk_ref, rw_ref):
    i = pl.program_id(0)
    bm = lcol_ref.shape[0]
    vj = lcol_ref[...]
    vi = lrow_ref[...]
    irow = i * bm + jax.lax.broadcasted_iota(jnp.int32, (bm, 1), 0)
    icol = jax.lax.broadcasted_iota(jnp.int32, (1, S), 1)
    gt = (vi > vj).astype(jnp.float32)
    eqlt = ((vi == vj) & (icol < irow)).astype(jnp.float32)
    rank = jnp.sum(gt + eqlt, axis=1, keepdims=True)
    tk = (rank < float(TOPK)).astype(jnp.float32)
    tk_ref[...] = tk
    rw_ref[...] = jax.nn.sigmoid(vj) * (1.0 - tk)


def _rank(logits_col, logits_row):
    bm = 512
    return pl.pallas_call(
        _rank_body,
        grid=(S // bm,),
        in_specs=[
            pl.BlockSpec((bm, 1), lambda i: (i, 0)),
            pl.BlockSpec((1, S), lambda i: (0, 0)),
        ],
        out_specs=[
            pl.BlockSpec((bm, 1), lambda i: (i, 0)),
            pl.BlockSpec((bm, 1), lambda i: (i, 0)),
        ],
        out_shape=[
            jax.ShapeDtypeStruct((S, 1), jnp.float32),
            jax.ShapeDtypeStruct((S, 1), jnp.float32),
        ],
    )(logits_col, logits_row)


def _mm_body(x_ref, w_ref, o_ref):
    @pl.when(pl.program_id(2) == 0)
    def _():
        o_ref[...] = jnp.zeros_like(o_ref)

    o_ref[...] += _dot(x_ref[...], w_ref[...])


def _matmul(x, w, bm, bn, bk):
    m, k = x.shape
    _, n = w.shape
    return pl.pallas_call(
        _mm_body,
        grid=(m // bm, n // bn, k // bk),
        in_specs=[
            pl.BlockSpec((bm, bk), lambda i, j, kk: (i, kk)),
            pl.BlockSpec((bk, bn), lambda i, j, kk: (kk, j)),
        ],
        out_specs=pl.BlockSpec((bm, bn), lambda i, j, kk: (i, j)),
        out_shape=jax.ShapeDtypeStruct((m, n), jnp.float32),
        compiler_params=pltpu.CompilerParams(
            dimension_semantics=("parallel", "parallel", "arbitrary")),
    )(x, w)


def _mm2_body(x_ref, wk_ref, wv_ref, ok_ref, ov_ref):
    @pl.when(pl.program_id(1) == 0)
    def _():
        ok_ref[...] = jnp.zeros_like(ok_ref)
        ov_ref[...] = jnp.zeros_like(ov_ref)

    xb = x_ref[...]
    ok_ref[...] += _dot(xb, wk_ref[...])
    ov_ref[...] += _dot(xb, wv_ref[...])


def _matmul_kv(x, Wk, Wv):
    bm, bk = 1024, 512
    n = NKV * HD
    return pl.pallas_call(
        _mm2_body,
        grid=(S // bm, HID // bk),
        in_specs=[
            pl.BlockSpec((bm, bk), lambda i, kk: (i, kk)),
            pl.BlockSpec((bk, n), lambda i, kk: (kk, 0)),
            pl.BlockSpec((bk, n), lambda i, kk: (kk, 0)),
        ],
        out_specs=[
            pl.BlockSpec((bm, n), lambda i, kk: (i, 0)),
            pl.BlockSpec((bm, n), lambda i, kk: (i, 0)),
        ],
        out_shape=[
            jax.ShapeDtypeStruct((S, n), jnp.float32),
            jax.ShapeDtypeStruct((S, n), jnp.float32),
        ],
        compiler_params=pltpu.CompilerParams(
            dimension_semantics=("parallel", "arbitrary")),
    )(x, Wk, Wv)


def _rmsnorm(x3, g):
    ms = jnp.mean(x3 * x3, axis=-1, keepdims=True)
    return x3 * jax.lax.rsqrt(ms + 1e-6) * g


def _rope(x3, cosb, sinb):
    x1 = x3[..., : HD // 2]
    x2 = x3[..., HD // 2:]
    h1 = x1 * cosb[..., : HD // 2] - x2 * sinb[..., : HD // 2]
    h2 = x2 * cosb[..., HD // 2:] + x1 * sinb[..., HD // 2:]
    return jnp.concatenate([h1, h2], axis=-1)


def _epi_body2(k_ref, v_ref, cos_ref, sin_ref, gk_ref, rw_ref,
               ko_ref, kv_ref):
    bm = k_ref.shape[0]
    cosb = cos_ref[...][:, None, :]
    sinb = sin_ref[...][:, None, :]
    kn = _rmsnorm(k_ref[...], gk_ref[...])
    ko_ref[...] = _rope(kn, cosb, sinb).reshape(bm, NKV * HD)
    rw = rw_ref[...]
    kv_ref[:, : NKV * HD] = kn.reshape(bm, NKV * HD) * rw
    kv_ref[:, NKV * HD:] = v_ref[...].reshape(bm, NKV * HD) * rw


def _epilogue(k, v, cos, sin, gk, restw):
    bm = 512
    return pl.pallas_call(
        _epi_body2,
        grid=(S // bm,),
        in_specs=[
            pl.BlockSpec((bm, NKV, HD), lambda i: (i, 0, 0)),
            pl.BlockSpec((bm, NKV, HD), lambda i: (i, 0, 0)),
            pl.BlockSpec((bm, HD), lambda i: (i, 0)),
            pl.BlockSpec((bm, HD), lambda i: (i, 0)),
            pl.BlockSpec((1, 1, HD), lambda i: (0, 0, 0)),
            pl.BlockSpec((bm, 1), lambda i: (i, 0)),
        ],
        out_specs=[
            pl.BlockSpec((bm, NKV * HD), lambda i: (i, 0)),
            pl.BlockSpec((bm, 2 * NKV * HD), lambda i: (i, 0)),
        ],
        out_shape=[
            jax.ShapeDtypeStruct((S, NKV * HD), jnp.float32),
            jax.ShapeDtypeStruct((S, 2 * NKV * HD), jnp.float32),
        ],
    )(k.reshape(S, NKV, HD), v.reshape(S, NKV, HD),
      cos, sin, gk.reshape(1, 1, HD), restw)


def _sk_body(m_ref, x_ref, o_ref):
    @pl.when(pl.program_id(0) == 0)
    def _():
        o_ref[...] = jnp.zeros_like(o_ref)

    o_ref[...] += _dot(m_ref[...], x_ref[...])


def _sketch(M, kvw):
    bk = 1024
    return pl.pallas_call(
        _sk_body,
        grid=(S // bk,),
        in_specs=[
            pl.BlockSpec((SK, bk), lambda kk: (0, kk)),
            pl.BlockSpec((bk, 2 * NKV * HD), lambda kk: (kk, 0)),
        ],
        out_specs=pl.BlockSpec((SK, 2 * NKV * HD), lambda kk: (0, 0)),
        out_shape=jax.ShapeDtypeStruct((SK, 2 * NKV * HD), jnp.float32),
        compiler_params=pltpu.CompilerParams(
            dimension_semantics=("arbitrary",)),
    )(M, kvw)


def _sc_gather(k_rope, v, idx):
    info = plsc.get_sparse_core_info()
    nc, ns = info.num_cores, info.num_subcores
    nw = nc * ns
    bpw = TOPK // nw
    d = NKV * HD
    mesh = plsc.VectorSubcoreMesh(core_axis_name="c", subcore_axis_name="s")

    @functools.partial(
        pl.kernel, mesh=mesh,
        out_type=[
            jax.ShapeDtypeStruct((TOPK, d), jnp.float32),
            jax.ShapeDtypeStruct((TOPK, d), jnp.float32),
        ],
        scratch_types=[
            pltpu.VMEM((bpw,), jnp.int32),
            pltpu.VMEM((bpw, d), jnp.float32),
            pltpu.SemaphoreType.DMA,
        ],
    )
    def g(k_hbm, v_hbm, idx_hbm, ok_hbm, ov_hbm, idx_v, rows_v, sem):
        wid = lax.axis_index("s") * nc + lax.axis_index("c")
        base = wid * bpw
        pltpu.sync_copy(idx_hbm.at[pl.ds(base, bpw)], idx_v)
        pltpu.async_copy(k_hbm.at[idx_v], rows_v, sem).wait()
        pltpu.sync_copy(rows_v, ok_hbm.at[pl.ds(base, bpw)])
        pltpu.async_copy(v_hbm.at[idx_v], rows_v, sem).wait()
        pltpu.sync_copy(rows_v, ov_hbm.at[pl.ds(base, bpw)])

    return g(k_rope, v, idx)


def _attn_body(nb_ref, q_ref, k_ref, v_ref, sk_ref, sv_ref, idx_ref, cos_ref,
               sin_ref, gq_ref, o_ref, *, bm, bkk):
    i = pl.program_id(1)
    scale = 1.0 / math.sqrt(HD)
    cosq = cos_ref[...]
    sinq = sin_ref[...]
    qpos = i * bm + jax.lax.broadcasted_iota(jnp.int32, (bm, 1), 0)
    qpos_f = qpos.astype(jnp.float32)

    ratio = jnp.float32((CA * CB) / SK)
    stimes = jax.lax.broadcasted_iota(
        jnp.int32, (1, SK), 1).astype(jnp.float32) * ratio
    smask = stimes <= qpos_f
    skb = sk_ref[...]
    svb = sv_ref[...]

    for hh in range(GQ):
        qh = q_ref[:, hh * HD:(hh + 1) * HD]
        qn = qh * jax.lax.rsqrt(
            jnp.mean(qh * qh, axis=1, keepdims=True) + 1e-6) * gq_ref[...]
        q1 = qn[:, : HD // 2]
        q2 = qn[:, HD // 2:]
        q = jnp.concatenate(
            [q1 * cosq[:, : HD // 2] - q2 * sinq[:, : HD // 2],
             q2 * cosq[:, HD // 2:] + q1 * sinq[:, HD // 2:]],
            axis=1) * scale

        def body(j, carry):
            m, l, acc = carry
            kb = k_ref[pl.ds(j * bkk, bkk), :]
            vb = v_ref[pl.ds(j * bkk, bkk), :]
            kpos = idx_ref[:, pl.ds(j * bkk, bkk)]
            okb = kpos <= qpos
            lg = _dot(q, kb, trans_b=True)
            lg = jnp.where(okb, lg, _NEG)
            mn = jnp.maximum(m, jnp.max(lg, axis=1, keepdims=True))
            p = jnp.where(okb, jnp.exp(lg - mn), 0.0)
            alpha = jnp.exp(m - mn)
            l = l * alpha + jnp.sum(p, axis=1, keepdims=True)
            acc = acc * alpha + _dot(p, vb)
            return mn, l, acc

        m0 = jnp.full((bm, 1), _NEG, jnp.float32)
        l0 = jnp.zeros((bm, 1), jnp.float32)
        a0 = jnp.zeros((bm, HD), jnp.float32)
        m, l, acc = jax.lax.fori_loop(0, nb_ref[0, i], body, (m0, l0, a0))

        lg = _dot(q, skb, trans_b=True)
        lg = jnp.where(smask, lg, _NEG)
        mn = jnp.maximum(m, jnp.max(lg, axis=1, keepdims=True))
        p = jnp.where(smask, jnp.exp(lg - mn), 0.0)
        alpha = jnp.exp(m - mn)
        l = l * alpha + jnp.sum(p, axis=1, keepdims=True)
        acc = acc * alpha + _dot(p, svb)
        o_ref[:, hh * HD:(hh + 1) * HD] = acc / l


def _attention(q, kd, vd, sketch_kv, idx_row, nblocks, cos, sin, gq):
    bm = 512
    bkk = 512
    return pl.pallas_call(
        functools.partial(_attn_body, bm=bm, bkk=bkk),
        grid=(NKV, S // bm),
        in_specs=[
            pl.BlockSpec(memory_space=pltpu.SMEM),
            pl.BlockSpec((bm, GQ * HD), lambda h, i: (i, h)),
            pl.BlockSpec((TOPK, HD), lambda h, i: (0, h)),
            pl.BlockSpec((TOPK, HD), lambda h, i: (0, h)),
            pl.BlockSpec((SK, HD), lambda h, i: (0, h)),
            pl.BlockSpec((SK, HD), lambda h, i: (0, NKV + h)),
            pl.BlockSpec((1, TOPK), lambda h, i: (0, 0)),
            pl.BlockSpec((bm, HD), lambda h, i: (i, 0)),
            pl.BlockSpec((bm, HD), lambda h, i: (i, 0)),
            pl.BlockSpec((1, HD), lambda h, i: (0, 0)),
        ],
        out_specs=pl.BlockSpec((bm, GQ * HD), lambda h, i: (i, h)),
        out_shape=jax.ShapeDtypeStruct((S, NH * HD), jnp.float32),
        compiler_params=pltpu.CompilerParams(
            dimension_semantics=("arbitrary", "arbitrary")),
    )(nblocks, q, kd, vd, sketch_kv, sketch_kv, idx_row, cos, sin, gq)


def _causal_mask(rows, cols):
    r = jnp.arange(rows)[:, None]
    c = jnp.arange(cols)[None, :]
    return (c < ((r + 1) * cols) // rows).astype(jnp.float32)


def _fold_sketch_matrix(kron_a, kron_b, sketch_scale):
    cka = kron_a * _causal_mask(RA, CA)
    ckb = kron_b * _causal_mask(RB, CB)
    nca = S // CB
    m4 = cka[None, :, :nca, None] * ckb[:, None, None, :]
    return m4.reshape(RB * RA, nca * CB) * sketch_scale[0]


def _freqs(position_embeddings):
    inv = 1.0 / (10000.0 ** (jnp.arange(0, HD, 2, dtype=jnp.float32) / HD))
    f = position_embeddings[0].astype(jnp.float32)[:, None] * inv[None, :]
    emb = jnp.concatenate([f, f], axis=-1)
    return jnp.cos(emb), jnp.sin(emb)


def kernel(hidden_states, position_embeddings, Wq, Wk, Wv, Wo, gq, gk,
           kron_a, kron_b, sketch_scale, W1, b1, W2, b2):
    x = hidden_states.reshape(S, HID)
    cos, sin = _freqs(position_embeddings)
    M = _fold_sketch_matrix(kron_a, kron_b, sketch_scale)

    imp_logits = jnp.tanh(hidden_states @ W1 + b1) @ W2 + b2
    imp_logits = imp_logits - math.log(S / SK)
    logits = imp_logits.reshape(S, 1)
    topk_col, restw = _rank(logits, logits.reshape(1, S))
    q = _matmul(x, Wq, bm=1024, bn=1024, bk=1024)
    k, v = _matmul_kv(x, Wk, Wv)
    k_rope, kvw = _epilogue(k, v, cos, sin, gk, restw)
    sketch_kv = _sketch(M, kvw)

    tkb = topk_col.reshape(S) > 0.0
    csum = jnp.cumsum(tkb.astype(jnp.int32))
    idx_sorted = jnp.zeros((TOPK,), jnp.int32).at[
        jnp.where(tkb, csum - 1, TOPK)].set(
        jnp.arange(S, dtype=jnp.int32), mode='drop')
    cnt = csum[511::512]
    nblocks = ((cnt + 511) // 512).reshape(1, S // 512)

    kd, vd = _sc_gather(k_rope, v, idx_sorted)
    attn = _attention(q, kd, vd, sketch_kv, idx_sorted.reshape(1, TOPK),
                      nblocks, cos, sin, gq.reshape(1, HD))
    out = _matmul(attn, Wo, bm=1024, bn=1024, bk=1024)
    return out.reshape(B, S, HID)
